# baseline (speedup 1.0000x reference)
import functools

import jax
import jax.numpy as jnp
from jax import lax
from jax.experimental import pallas as pl
from jax.experimental.pallas import tpu as pltpu
from jax.experimental.pallas import tpu_sc as plsc

_NUM_CORES = 2
_NUM_SUBCORES = 16
_NW = _NUM_CORES * _NUM_SUBCORES
_CHUNK = 32


@functools.partial(jax.jit, static_argnames=("b", "l", "d"))
def _sc_embed(tok, table, pe, *, b, l, d):
    n = b * l
    per_w = n // _NW
    n_chunks = per_w // _CHUNK
    assert n_chunks >= 4 and n_chunks % 2 == 0
    n_pairs = n_chunks // 2
    nvec = d // 16

    mesh = plsc.VectorSubcoreMesh(
        core_axis_name="c", subcore_axis_name="s",
        num_cores=_NUM_CORES, num_subcores=_NUM_SUBCORES,
    )

    @functools.partial(
        pl.kernel,
        mesh=mesh,
        out_type=jax.ShapeDtypeStruct((b, l, d), jnp.float32),
        scratch_types=[
            pltpu.VMEM((per_w,), jnp.int32),
            [pltpu.VMEM((_CHUNK, d), jnp.float32) for _ in range(2)],
            [pltpu.VMEM((_CHUNK, d), jnp.float32) for _ in range(2)],
            [pltpu.SemaphoreType.DMA for _ in range(2)],
            [pltpu.SemaphoreType.DMA for _ in range(2)],
            [pltpu.SemaphoreType.DMA for _ in range(2)],
        ],
    )
    def k(tok_hbm, table_hbm, pe_hbm, out_hbm,
          idx_v, rows, peb, sem_g, sem_p, sem_s):
        wid = lax.axis_index("s") * _NUM_CORES + lax.axis_index("c")
        bi = wid * per_w // l
        l0 = lax.rem(wid * per_w, l)

        def gather(c, p):
            pltpu.async_copy(
                table_hbm.at[idx_v.at[pl.ds(c * _CHUNK, _CHUNK)]],
                rows[p], sem_g[p])

        def pe_load(c, p):
            pltpu.async_copy(
                pe_hbm.at[0, pl.ds(l0 + c * _CHUNK, _CHUNK), :],
                peb[p], sem_p[p])

        def store(c, p):
            pltpu.async_copy(
                rows[p], out_hbm.at[bi, pl.ds(l0 + c * _CHUNK, _CHUNK), :],
                sem_s[p])

        def wait_gather(p):
            pltpu.make_async_copy(
                pe_hbm.at[0, pl.ds(l0, _CHUNK), :], rows[p], sem_g[p]).wait()

        def wait_pe(p):
            pltpu.make_async_copy(
                pe_hbm.at[0, pl.ds(l0, _CHUNK), :], peb[p], sem_p[p]).wait()

        def wait_store(p):
            pltpu.make_async_copy(
                rows[p], out_hbm.at[bi, pl.ds(l0, _CHUNK), :],
                sem_s[p]).wait()

        def add(p_row, p_pe):
            def add_row(r, _):
                for j in range(nvec):
                    sl = pl.ds(j * 16, 16)
                    plsc.addupdate(rows[p_row].at[r, sl], peb[p_pe][r, sl])
                return _
            lax.fori_loop(0, _CHUNK, add_row, 0, unroll=False)

        pltpu.sync_copy(tok_hbm.at[bi, pl.ds(l0, per_w)], idx_v)
        gather(0, 0)
        gather(1, 1)
        pe_load(0, 0)
        pe_load(1, 1)
        wait_gather(0)
        wait_pe(0)
        add(0, 0)
        pe_load(2, 0)
        store(0, 0)
        wait_gather(1)
        wait_pe(1)
        add(1, 1)
        pe_load(3, 1)
        store(1, 1)
        wait_store(0)
        gather(2, 0)

        def pair(g, _):
            c0 = 2 * g
            wait_store(1)
            gather(c0 + 1, 1)
            wait_gather(0)
            wait_pe(0)
            add(0, 0)
            pe_load(c0 + 2, 0)
            store(c0, 0)
            wait_gather(1)
            wait_pe(1)
            add(1, 1)
            pe_load(c0 + 3, 1)
            store(c0 + 1, 1)
            wait_store(0)
            gather(c0 + 2, 0)
            return _

        lax.fori_loop(1, n_pairs - 1, pair, 0, unroll=False)

        c0 = n_chunks - 2
        wait_store(1)
        gather(c0 + 1, 1)
        wait_gather(0)
        wait_pe(0)
        add(0, 0)
        store(c0, 0)
        wait_gather(1)
        wait_pe(1)
        add(1, 1)
        store(c0 + 1, 1)
        wait_store(0)
        wait_store(1)

    return k(tok, table, pe)


def kernel(tokens, table, pe):
    b, l = tokens.shape
    d = table.shape[1]
    return _sc_embed(tokens, table, pe, b=b, l=l, d=d)


# speedup vs baseline: 1.0067x; 1.0067x over previous
import functools

import jax
import jax.numpy as jnp
from jax import lax
from jax.experimental import pallas as pl
from jax.experimental.pallas import tpu as pltpu
from jax.experimental.pallas import tpu_sc as plsc

_NUM_CORES = 2
_NUM_SUBCORES = 16
_NW = _NUM_CORES * _NUM_SUBCORES
_CHUNK = 16
_NROWBUF = 6
_NPEBUF = 4
_GA = 4


@functools.partial(jax.jit, static_argnames=("b", "l", "d"))
def _sc_embed(tok, table, pe, *, b, l, d):
    n = b * l
    per_w = n // _NW
    n_chunks = per_w // _CHUNK
    nvec = d // 16

    mesh = plsc.VectorSubcoreMesh(
        core_axis_name="c", subcore_axis_name="s",
        num_cores=_NUM_CORES, num_subcores=_NUM_SUBCORES,
    )

    @functools.partial(
        pl.kernel,
        mesh=mesh,
        out_type=jax.ShapeDtypeStruct((b, l, d), jnp.float32),
        scratch_types=[
            pltpu.VMEM((per_w,), jnp.int32),
            [pltpu.VMEM((_CHUNK, d), jnp.float32) for _ in range(_NROWBUF)],
            [pltpu.VMEM((_CHUNK, d), jnp.float32) for _ in range(_NPEBUF)],
            [pltpu.SemaphoreType.DMA for _ in range(_NROWBUF)],
            [pltpu.SemaphoreType.DMA for _ in range(_NPEBUF)],
            [pltpu.SemaphoreType.DMA for _ in range(_NROWBUF)],
        ],
    )
    def k(tok_hbm, table_hbm, pe_hbm, out_hbm,
          idx_v, rows, peb, sem_g, sem_p, sem_s):
        wid = lax.axis_index("s") * _NUM_CORES + lax.axis_index("c")
        bi = wid * per_w // l
        l0 = lax.rem(wid * per_w, l)

        def gather(c):
            return pltpu.async_copy(
                table_hbm.at[idx_v.at[pl.ds(c * _CHUNK, _CHUNK)]],
                rows[c % _NROWBUF], sem_g[c % _NROWBUF])

        def pe_load(c):
            return pltpu.async_copy(
                pe_hbm.at[0, pl.ds(l0 + c * _CHUNK, _CHUNK), :],
                peb[c % _NPEBUF], sem_p[c % _NPEBUF])

        pltpu.sync_copy(tok_hbm.at[bi, pl.ds(l0, per_w)], idx_v)
        pend_g = {c: gather(c) for c in range(min(_GA, n_chunks))}
        pend_p = {c: pe_load(c) for c in range(min(_NPEBUF, n_chunks))}
        pend_s = {}

        for c in range(n_chunks):
            rb, pb = c % _NROWBUF, c % _NPEBUF
            if c + _GA < n_chunks:
                if c + _GA - _NROWBUF >= 0:
                    pend_s.pop(c + _GA - _NROWBUF).wait()
                pend_g[c + _GA] = gather(c + _GA)
            pend_g.pop(c).wait()
            pend_p.pop(c).wait()

            def add_row(r, _):
                for j in range(nvec):
                    sl = pl.ds(j * 16, 16)
                    plsc.addupdate(rows[rb].at[r, sl], peb[pb][r, sl])
                return _

            lax.fori_loop(0, _CHUNK, add_row, 0, unroll=False)
            if c + _NPEBUF < n_chunks:
                pend_p[c + _NPEBUF] = pe_load(c + _NPEBUF)
            pend_s[c] = pltpu.async_copy(
                rows[rb], out_hbm.at[bi, pl.ds(l0 + c * _CHUNK, _CHUNK), :],
                sem_s[rb])
        for c in sorted(pend_s):
            pend_s[c].wait()

    return k(tok, table, pe)


def kernel(tokens, table, pe):
    b, l = tokens.shape
    d = table.shape[1]
    return _sc_embed(tokens, table, pe, b=b, l=l, d=d)
